# Initial kernel scaffold; baseline (speedup 1.0000x reference)
import jax
import jax.numpy as jnp
from jax.experimental import pallas as pl


def kernel(label_ids, local2global, table):
    raise NotImplementedError("write your pallas kernel here")


# speedup vs baseline: 10.3984x; 10.3984x over previous
import functools

import jax
import jax.numpy as jnp
from jax import lax
from jax.experimental import pallas as pl
from jax.experimental.pallas import tpu as pltpu
from jax.experimental.pallas import tpu_sc as plsc

EMB = 64
NC = 2
NS = 16
NW = NC * NS
CHUNK = 128
GRP = 4


@functools.lru_cache(maxsize=None)
def _build(N):
    n_per_w = N // NW
    n_chunks = n_per_w // CHUNK
    n_groups = n_chunks // GRP
    mesh = plsc.VectorSubcoreMesh(core_axis_name="c", subcore_axis_name="s")

    @functools.partial(
        pl.kernel,
        mesh=mesh,
        compiler_params=pltpu.CompilerParams(use_tc_tiling_on_sc=False),
        out_type=jax.ShapeDtypeStruct((N, EMB), jnp.float32),
        scratch_types=[
            pltpu.VMEM((n_chunks, CHUNK), jnp.int32),
            pltpu.VMEM((n_chunks, CHUNK), jnp.int32),
            pltpu.VMEM((GRP, CHUNK, EMB), jnp.float32),
            pltpu.SemaphoreType.DMA,
        ],
    )
    def emb_kernel(labels_hbm, l2g_hbm, table_hbm, out_hbm,
                   idx_v, gidx_v, rows_v, gsem):
        wid = lax.axis_index("s") * NC + lax.axis_index("c")
        base = wid * n_per_w

        pltpu.sync_copy(labels_hbm.at[wid], idx_v)

        def gidx_body(jj, carry):
            handles = []
            for b in range(GRP):
                j = jj * GRP + b
                handles.append(
                    pltpu.async_copy(l2g_hbm.at[idx_v.at[j]], gidx_v.at[j],
                                     gsem))
            for h in handles:
                h.wait()
            return carry

        lax.fori_loop(0, n_groups, gidx_body, 0, unroll=False)

        def row_body(jj, carry):
            handles = []
            for b in range(GRP):
                j = jj * GRP + b
                handles.append(
                    pltpu.async_copy(table_hbm.at[gidx_v.at[j]], rows_v.at[b],
                                     gsem))
            for b in range(GRP):
                j = jj * GRP + b
                handles[b].wait()
                pltpu.sync_copy(rows_v.at[b],
                                out_hbm.at[pl.ds(base + j * CHUNK, CHUNK)])
            return carry

        lax.fori_loop(0, n_groups, row_body, 0, unroll=False)

    return emb_kernel


def kernel(label_ids, local2global, table):
    B, L = label_ids.shape
    N = B * L
    labels = label_ids.reshape(NW, N // NW // CHUNK, CHUNK)
    out = _build(N)(labels, local2global, table)
    return out.reshape(B, L, EMB)


# speedup vs baseline: 10.5210x; 1.0118x over previous
import functools

import jax
import jax.numpy as jnp
from jax import lax
from jax.experimental import pallas as pl
from jax.experimental.pallas import tpu as pltpu
from jax.experimental.pallas import tpu_sc as plsc

EMB = 64
NC = 2
NS = 16
NW = NC * NS
BCHUNK = 4
GRP = 8


@functools.lru_cache(maxsize=None)
def _build(B, L):
    N = B * L
    CHUNK = BCHUNK * L
    b_per_w = B // NW
    n_per_w = b_per_w * L
    n_chunks = b_per_w // BCHUNK
    n_groups = n_chunks // GRP
    mesh = plsc.VectorSubcoreMesh(core_axis_name="c", subcore_axis_name="s")

    @functools.partial(
        pl.kernel,
        mesh=mesh,
        compiler_params=pltpu.CompilerParams(use_tc_tiling_on_sc=False),
        out_type=jax.ShapeDtypeStruct((B, L, EMB), jnp.float32),
        scratch_types=[
            pltpu.VMEM((n_chunks, CHUNK), jnp.int32),
            pltpu.VMEM((n_chunks, CHUNK), jnp.int32),
            pltpu.VMEM((GRP, CHUNK, EMB), jnp.float32),
            pltpu.SemaphoreType.DMA,
            pltpu.SemaphoreType.DMA,
        ],
    )
    def emb_kernel(labels_hbm, l2g_hbm, table_hbm, out_hbm,
                   idx_v, gidx_v, rows_v, gsem, osem):
        wid = lax.axis_index("s") * NC + lax.axis_index("c")
        b_base = wid * b_per_w

        pltpu.sync_copy(labels_hbm.at[wid], idx_v)

        def gidx_body(jj, carry):
            handles = []
            for b in range(GRP):
                j = jj * GRP + b
                handles.append(
                    pltpu.async_copy(l2g_hbm.at[idx_v.at[j]], gidx_v.at[j],
                                     gsem))
            for h in handles:
                h.wait()
            return carry

        lax.fori_loop(0, n_groups, gidx_body, 0, unroll=False)

        def row_body(jj, carry):
            @pl.when(jj > 0)
            def _drain_prev():
                for b in range(GRP):
                    for k in range(BCHUNK):
                        pltpu.make_async_copy(
                            rows_v.at[b, pl.ds(k * L, L)],
                            out_hbm.at[b_base], osem).wait()

            handles = []
            for b in range(GRP):
                j = jj * GRP + b
                handles.append(
                    pltpu.async_copy(table_hbm.at[gidx_v.at[j]], rows_v.at[b],
                                     gsem))
            for b in range(GRP):
                j = jj * GRP + b
                handles[b].wait()
                for k in range(BCHUNK):
                    pltpu.async_copy(
                        rows_v.at[b, pl.ds(k * L, L)],
                        out_hbm.at[b_base + j * BCHUNK + k], osem)
            return carry

        lax.fori_loop(0, n_groups, row_body, 0, unroll=False)
        for b in range(GRP):
            for k in range(BCHUNK):
                pltpu.make_async_copy(
                    rows_v.at[b, pl.ds(k * L, L)],
                    out_hbm.at[b_base], osem).wait()

    return emb_kernel


def kernel(label_ids, local2global, table):
    B, L = label_ids.shape
    CHUNK = BCHUNK * L
    labels = label_ids.reshape(NW, B // NW // BCHUNK, CHUNK)
    return _build(B, L)(labels, local2global, table)
